# baseline (speedup 1.0000x reference)
import functools

import jax
import jax.numpy as jnp
from jax import lax
from jax.experimental import pallas as pl
from jax.experimental.pallas import tpu as pltpu
from jax.experimental.pallas import tpu_sc as plsc

SIZE = 128
NSCALE = 4
D = NSCALE * SIZE

NC = 2
NS = 16
NW = NC * NS

CHUNK = 112
RING = 7


def _sc_gather(n_pad, nch):
    ring = min(RING, nch)
    assert nch % ring == 0
    rows_per_worker = nch * CHUNK
    mesh = plsc.VectorSubcoreMesh(
        core_axis_name="c", subcore_axis_name="s",
        num_cores=NC, num_subcores=NS)

    @functools.partial(
        pl.kernel,
        out_type=jax.ShapeDtypeStruct((n_pad, D), jnp.float32),
        mesh=mesh,
        scratch_types=(
            [pltpu.VMEM((nch, CHUNK), jnp.int32) for _ in range(NSCALE)]
            + [pltpu.VMEM((CHUNK, SIZE), jnp.float32) for _ in range(ring)]
            + [pltpu.SemaphoreType.DMA for _ in range(NSCALE)]
            + [pltpu.SemaphoreType.DMA for _ in range(ring)]
            + [pltpu.SemaphoreType.DMA for _ in range(ring)]
        ),
    )
    def sc_kernel(*refs):
        xs = refs[0:NSCALE]
        ss = refs[NSCALE:2 * NSCALE]
        o = refs[2 * NSCALE]
        rest = refs[2 * NSCALE + 1:]
        idxs = rest[0:NSCALE]
        bufs = rest[NSCALE:NSCALE + ring]
        isems = rest[NSCALE + ring:2 * NSCALE + ring]
        gsems = rest[2 * NSCALE + ring:2 * NSCALE + 2 * ring]
        ssems = rest[2 * NSCALE + 2 * ring:2 * NSCALE + 3 * ring]

        wid = lax.axis_index("s") * NC + lax.axis_index("c")
        base = pl.multiple_of(wid * rows_per_worker, 8)

        for t in range(NSCALE):
            pltpu.make_async_copy(ss[t].at[wid], idxs[t], isems[t]).start()

        for t in range(NSCALE):
            x, idx = xs[t], idxs[t]

            def gather(c, r, x=x, idx=idx):
                return pltpu.make_async_copy(x.at[idx.at[c]], bufs[r],
                                             gsems[r])

            def store(c, r, t=t):
                row = pl.multiple_of(base + c * CHUNK, 8)
                return pltpu.make_async_copy(
                    bufs[r],
                    o.at[pl.ds(row, CHUNK), pl.ds(t * SIZE, SIZE)],
                    ssems[r])

            pltpu.make_async_copy(ss[t].at[wid], idx, isems[t]).wait()
            for r in range(ring):
                gather(r, r).start()

            def body(j0, _, gather=gather, store=store):
                c0 = j0 * ring
                for r in range(ring):
                    gather(c0 + r, r).wait()
                    store(c0 + r, r).start()
                for r in range(ring):
                    store(c0 + r, r).wait()
                    gather(c0 + r + ring, r).start()
                return ()

            lax.fori_loop(0, nch // ring - 1, body, (), unroll=False)

            c0 = nch - ring
            for r in range(ring):
                gather(c0 + r, r).wait()
                store(c0 + r, r).start()
            for r in range(ring):
                store(c0 + r, r).wait()

    return sc_kernel


def _tc_compute(g, w, b, gamma, beta, o):
    dn = (((1,), (1,)), ((), ()))
    acc = lax.dot_general(g[...], w[...], dn,
                          preferred_element_type=jnp.float32)
    y = jnp.maximum(acc + b[...], 0.0)
    mean = jnp.mean(y, axis=1, keepdims=True)
    yc = y - mean
    var = jnp.mean(yc * yc, axis=1, keepdims=True)
    o[...] = yc * lax.rsqrt(var + 1e-6) * gamma[...] + beta[...]


SLAB_NCH = (2, 5, 7, 14)
BN = 896


def kernel(x_friend, x_follow, x_group, x_event,
           src_friend, src_follow, src_group, src_event,
           W, b, gamma, beta):
    n = x_friend.shape[0]
    rows_w = NW * CHUNK
    nch_total = sum(SLAB_NCH)
    n_pad = nch_total * rows_w
    assert n_pad >= n
    pad = n_pad - n

    def prep(s):
        return jnp.concatenate([s, jnp.zeros((pad,), jnp.int32)])

    srcs = [prep(s) for s in (src_friend, src_follow, src_group, src_event)]
    xs = [x_friend, x_follow, x_group, x_event]
    b2, gamma2, beta2 = b.reshape(1, D), gamma.reshape(1, D), beta.reshape(1, D)

    gs = []
    off = 0
    for nch in SLAB_NCH:
        rows_slab = nch * rows_w
        slab_srcs = [s[off:off + rows_slab].reshape(NW, nch, CHUNK)
                     for s in srcs]
        gs.append(_sc_gather(rows_slab, nch)(*xs, *slab_srcs))
        off += rows_slab

    common_specs = [
        pl.BlockSpec((BN, D), lambda i: (i, 0)),
        pl.BlockSpec((D, D), lambda i: (0, 0)),
        pl.BlockSpec((1, D), lambda i: (0, 0)),
        pl.BlockSpec((1, D), lambda i: (0, 0)),
        pl.BlockSpec((1, D), lambda i: (0, 0)),
    ]

    out = None
    off_blocks = 0
    for k, nch in enumerate(SLAB_NCH):
        rows_slab = nch * rows_w
        assert rows_slab % BN == 0, (rows_slab, BN)
        blocks_s = rows_slab // BN

        def out_map(i, ob=off_blocks):
            return (ob + i, 0)
        if k == 0:
            out = pl.pallas_call(
                _tc_compute,
                grid=(blocks_s,),
                in_specs=common_specs,
                out_specs=pl.BlockSpec((BN, D), out_map),
                out_shape=jax.ShapeDtypeStruct((n, D), jnp.float32),
            )(gs[k], W, b2, gamma2, beta2)
        else:
            def body(g, w, bb, gam, bet, prev, o):
                _tc_compute(g, w, bb, gam, bet, o)
            out = pl.pallas_call(
                body,
                grid=(blocks_s,),
                in_specs=common_specs
                + [pl.BlockSpec(memory_space=pltpu.MemorySpace.HBM)],
                out_specs=pl.BlockSpec((BN, D), out_map),
                out_shape=jax.ShapeDtypeStruct((n, D), jnp.float32),
                input_output_aliases={5: 0},
            )(gs[k], W, b2, gamma2, beta2, out)
        off_blocks += blocks_s
    return out


# speedup vs baseline: 1.0940x; 1.0940x over previous
import functools

import jax
import jax.numpy as jnp
from jax import lax
from jax.experimental import pallas as pl
from jax.experimental.pallas import tpu as pltpu
from jax.experimental.pallas import tpu_sc as plsc

SIZE = 128
NSCALE = 4
D = NSCALE * SIZE

NC = 2
NS = 16
NW = NC * NS

CHUNK = 112
RING = 7


def _sc_gather(n_pad, nch):
    ring = min(RING, nch)
    assert nch % ring == 0
    rows_per_worker = nch * CHUNK
    mesh = plsc.VectorSubcoreMesh(
        core_axis_name="c", subcore_axis_name="s",
        num_cores=NC, num_subcores=NS)

    @functools.partial(
        pl.kernel,
        out_type=jax.ShapeDtypeStruct((n_pad, D), jnp.float32),
        mesh=mesh,
        scratch_types=(
            [pltpu.VMEM((nch, CHUNK), jnp.int32) for _ in range(NSCALE)]
            + [pltpu.VMEM((CHUNK, SIZE), jnp.float32) for _ in range(ring)]
            + [pltpu.SemaphoreType.DMA for _ in range(NSCALE)]
            + [pltpu.SemaphoreType.DMA for _ in range(ring)]
            + [pltpu.SemaphoreType.DMA for _ in range(ring)]
        ),
    )
    def sc_kernel(*refs):
        xs = refs[0:NSCALE]
        ss = refs[NSCALE:2 * NSCALE]
        o = refs[2 * NSCALE]
        rest = refs[2 * NSCALE + 1:]
        idxs = rest[0:NSCALE]
        bufs = rest[NSCALE:NSCALE + ring]
        isems = rest[NSCALE + ring:2 * NSCALE + ring]
        gsems = rest[2 * NSCALE + ring:2 * NSCALE + 2 * ring]
        ssems = rest[2 * NSCALE + 2 * ring:2 * NSCALE + 3 * ring]

        wid = lax.axis_index("s") * NC + lax.axis_index("c")
        base = pl.multiple_of(wid * rows_per_worker, 8)

        for t in range(NSCALE):
            pltpu.make_async_copy(ss[t].at[wid], idxs[t], isems[t]).start()

        for t in range(NSCALE):
            x, idx = xs[t], idxs[t]

            def gather(c, r, x=x, idx=idx):
                return pltpu.make_async_copy(x.at[idx.at[c]], bufs[r],
                                             gsems[r])

            def store(c, r, t=t):
                row = pl.multiple_of(base + c * CHUNK, 8)
                return pltpu.make_async_copy(
                    bufs[r],
                    o.at[pl.ds(row, CHUNK), pl.ds(t * SIZE, SIZE)],
                    ssems[r])

            pltpu.make_async_copy(ss[t].at[wid], idx, isems[t]).wait()
            for r in range(ring):
                gather(r, r).start()

            def body(j0, _, gather=gather, store=store):
                c0 = j0 * ring
                for r in range(ring):
                    gather(c0 + r, r).wait()
                    store(c0 + r, r).start()
                for r in range(ring):
                    store(c0 + r, r).wait()
                    gather(c0 + r + ring, r).start()
                return ()

            lax.fori_loop(0, nch // ring - 1, body, (), unroll=False)

            c0 = nch - ring
            for r in range(ring):
                gather(c0 + r, r).wait()
                store(c0 + r, r).start()
            for r in range(ring):
                store(c0 + r, r).wait()

    return sc_kernel


def _tc_compute(g, w, b, gamma, beta, o):
    dn = (((1,), (1,)), ((), ()))
    acc = lax.dot_general(g[...], w[...], dn,
                          preferred_element_type=jnp.float32)
    y = jnp.maximum(acc + b[...], 0.0)
    mean = jnp.mean(y, axis=1, keepdims=True)
    yc = y - mean
    var = jnp.mean(yc * yc, axis=1, keepdims=True)
    o[...] = yc * lax.rsqrt(var + 1e-6) * gamma[...] + beta[...]


SLAB_NCH = (7, 7, 7, 7)
BN = 896


def kernel(x_friend, x_follow, x_group, x_event,
           src_friend, src_follow, src_group, src_event,
           W, b, gamma, beta):
    n = x_friend.shape[0]
    rows_w = NW * CHUNK
    nch_total = sum(SLAB_NCH)
    n_pad = nch_total * rows_w
    assert n_pad >= n
    pad = n_pad - n

    def prep(s):
        return jnp.concatenate([s, jnp.zeros((pad,), jnp.int32)])

    srcs = [prep(s) for s in (src_friend, src_follow, src_group, src_event)]
    xs = [x_friend, x_follow, x_group, x_event]
    b2, gamma2, beta2 = b.reshape(1, D), gamma.reshape(1, D), beta.reshape(1, D)

    gs = []
    off = 0
    for nch in SLAB_NCH:
        rows_slab = nch * rows_w
        slab_srcs = [s[off:off + rows_slab].reshape(NW, nch, CHUNK)
                     for s in srcs]
        gs.append(_sc_gather(rows_slab, nch)(*xs, *slab_srcs))
        off += rows_slab

    common_specs = [
        pl.BlockSpec((BN, D), lambda i: (i, 0)),
        pl.BlockSpec((D, D), lambda i: (0, 0)),
        pl.BlockSpec((1, D), lambda i: (0, 0)),
        pl.BlockSpec((1, D), lambda i: (0, 0)),
        pl.BlockSpec((1, D), lambda i: (0, 0)),
    ]

    out = None
    off_blocks = 0
    for k, nch in enumerate(SLAB_NCH):
        rows_slab = nch * rows_w
        assert rows_slab % BN == 0, (rows_slab, BN)
        blocks_s = rows_slab // BN

        def out_map(i, ob=off_blocks):
            return (ob + i, 0)
        if k == 0:
            out = pl.pallas_call(
                _tc_compute,
                grid=(blocks_s,),
                in_specs=common_specs,
                out_specs=pl.BlockSpec((BN, D), out_map),
                out_shape=jax.ShapeDtypeStruct((n, D), jnp.float32),
            )(gs[k], W, b2, gamma2, beta2)
        else:
            def body(g, w, bb, gam, bet, prev, o):
                _tc_compute(g, w, bb, gam, bet, o)
            out = pl.pallas_call(
                body,
                grid=(blocks_s,),
                in_specs=common_specs
                + [pl.BlockSpec(memory_space=pltpu.MemorySpace.HBM)],
                out_specs=pl.BlockSpec((BN, D), out_map),
                out_shape=jax.ShapeDtypeStruct((n, D), jnp.float32),
                input_output_aliases={5: 0},
            )(gs[k], W, b2, gamma2, beta2, out)
        off_blocks += blocks_s
    return out


# speedup vs baseline: 1.1738x; 1.0730x over previous
import functools

import jax
import jax.numpy as jnp
from jax import lax
from jax.experimental import pallas as pl
from jax.experimental.pallas import tpu as pltpu
from jax.experimental.pallas import tpu_sc as plsc

SIZE = 128
NSCALE = 4
D = NSCALE * SIZE

NC = 2
NS = 16
NW = NC * NS

CHUNK = 112
RING = 7


def _sc_gather(n_pad, nch):
    ring = min(RING, nch)
    assert nch % ring == 0
    rows_per_worker = nch * CHUNK
    mesh = plsc.VectorSubcoreMesh(
        core_axis_name="c", subcore_axis_name="s",
        num_cores=NC, num_subcores=NS)

    @functools.partial(
        pl.kernel,
        out_type=jax.ShapeDtypeStruct((n_pad, D), jnp.float32),
        mesh=mesh,
        scratch_types=(
            [pltpu.VMEM((nch, CHUNK), jnp.int32) for _ in range(NSCALE)]
            + [pltpu.VMEM((CHUNK, SIZE), jnp.float32) for _ in range(ring)]
            + [pltpu.SemaphoreType.DMA for _ in range(NSCALE)]
            + [pltpu.SemaphoreType.DMA for _ in range(ring)]
            + [pltpu.SemaphoreType.DMA for _ in range(ring)]
        ),
    )
    def sc_kernel(*refs):
        xs = refs[0:NSCALE]
        ss = refs[NSCALE:2 * NSCALE]
        o = refs[2 * NSCALE]
        rest = refs[2 * NSCALE + 1:]
        idxs = rest[0:NSCALE]
        bufs = rest[NSCALE:NSCALE + ring]
        isems = rest[NSCALE + ring:2 * NSCALE + ring]
        gsems = rest[2 * NSCALE + ring:2 * NSCALE + 2 * ring]
        ssems = rest[2 * NSCALE + 2 * ring:2 * NSCALE + 3 * ring]

        wid = lax.axis_index("s") * NC + lax.axis_index("c")
        base = pl.multiple_of(wid * rows_per_worker, 8)

        for t in range(NSCALE):
            pltpu.make_async_copy(ss[t].at[wid], idxs[t], isems[t]).start()

        for t in range(NSCALE):
            x, idx = xs[t], idxs[t]

            def gather(c, r, x=x, idx=idx):
                return pltpu.make_async_copy(x.at[idx.at[c]], bufs[r],
                                             gsems[r])

            def store(c, r, t=t):
                row = pl.multiple_of(base + c * CHUNK, 8)
                return pltpu.make_async_copy(
                    bufs[r],
                    o.at[pl.ds(row, CHUNK), pl.ds(t * SIZE, SIZE)],
                    ssems[r])

            pltpu.make_async_copy(ss[t].at[wid], idx, isems[t]).wait()
            for r in range(ring):
                gather(r, r).start()

            def body(j0, _, gather=gather, store=store):
                c0 = j0 * ring
                for r in range(ring):
                    gather(c0 + r, r).wait()
                    store(c0 + r, r).start()
                for r in range(ring):
                    store(c0 + r, r).wait()
                    gather(c0 + r + ring, r).start()
                return ()

            lax.fori_loop(0, nch // ring - 1, body, (), unroll=False)

            c0 = nch - ring
            for r in range(ring):
                gather(c0 + r, r).wait()
                store(c0 + r, r).start()
            for r in range(ring):
                store(c0 + r, r).wait()

    return sc_kernel


def _tc_compute(g, w, b, gamma, beta, o):
    dn = (((1,), (1,)), ((), ()))
    acc = lax.dot_general(g[...], w[...], dn,
                          preferred_element_type=jnp.float32)
    y = jnp.maximum(acc + b[...], 0.0)
    mean = jnp.mean(y, axis=1, keepdims=True)
    yc = y - mean
    var = jnp.mean(yc * yc, axis=1, keepdims=True)
    o[...] = yc * lax.rsqrt(var + 1e-6) * gamma[...] + beta[...]


SLAB_NCH = (7, 7, 7, 5, 2)
BN = 896


def kernel(x_friend, x_follow, x_group, x_event,
           src_friend, src_follow, src_group, src_event,
           W, b, gamma, beta):
    n = x_friend.shape[0]
    rows_w = NW * CHUNK
    nch_total = sum(SLAB_NCH)
    n_pad = nch_total * rows_w
    assert n_pad >= n
    pad = n_pad - n

    def prep(s):
        return jnp.concatenate([s, jnp.zeros((pad,), jnp.int32)])

    srcs = [prep(s) for s in (src_friend, src_follow, src_group, src_event)]
    xs = [x_friend, x_follow, x_group, x_event]
    b2, gamma2, beta2 = b.reshape(1, D), gamma.reshape(1, D), beta.reshape(1, D)

    gs = []
    off = 0
    for nch in SLAB_NCH:
        rows_slab = nch * rows_w
        slab_srcs = [s[off:off + rows_slab].reshape(NW, nch, CHUNK)
                     for s in srcs]
        gs.append(_sc_gather(rows_slab, nch)(*xs, *slab_srcs))
        off += rows_slab

    common_specs = [
        pl.BlockSpec((BN, D), lambda i: (i, 0)),
        pl.BlockSpec((D, D), lambda i: (0, 0)),
        pl.BlockSpec((1, D), lambda i: (0, 0)),
        pl.BlockSpec((1, D), lambda i: (0, 0)),
        pl.BlockSpec((1, D), lambda i: (0, 0)),
    ]

    out = None
    off_blocks = 0
    for k, nch in enumerate(SLAB_NCH):
        rows_slab = nch * rows_w
        assert rows_slab % BN == 0, (rows_slab, BN)
        blocks_s = rows_slab // BN

        def out_map(i, ob=off_blocks):
            return (ob + i, 0)
        if k == 0:
            out = pl.pallas_call(
                _tc_compute,
                grid=(blocks_s,),
                in_specs=common_specs,
                out_specs=pl.BlockSpec((BN, D), out_map),
                out_shape=jax.ShapeDtypeStruct((n, D), jnp.float32),
            )(gs[k], W, b2, gamma2, beta2)
        else:
            def body(g, w, bb, gam, bet, prev, o):
                _tc_compute(g, w, bb, gam, bet, o)
            out = pl.pallas_call(
                body,
                grid=(blocks_s,),
                in_specs=common_specs
                + [pl.BlockSpec(memory_space=pltpu.MemorySpace.HBM)],
                out_specs=pl.BlockSpec((BN, D), out_map),
                out_shape=jax.ShapeDtypeStruct((n, D), jnp.float32),
                input_output_aliases={5: 0},
            )(gs[k], W, b2, gamma2, beta2, out)
        off_blocks += blocks_s
    return out
